# Initial kernel scaffold; baseline (speedup 1.0000x reference)
import jax
import jax.numpy as jnp
from jax.experimental import pallas as pl


def kernel(x, W, b, edge_index):
    raise NotImplementedError("write your pallas kernel here")


# speedup vs baseline: 51.6437x; 51.6437x over previous
import jax
import jax.numpy as jnp
from jax import lax
from jax.experimental import pallas as pl


def _build_a_kernel(ei_ref, a_ref):
    V = a_ref.shape[0]
    E = ei_ref.shape[1]
    ei = ei_ref[...]
    row = ei[0:1, :]
    col = ei[1:2, :]
    ids = lax.broadcasted_iota(jnp.int32, (V, E), 0)
    C = (ids == col).astype(jnp.float32)
    R = (ids == row).astype(jnp.float32)
    deg = jnp.sum(C, axis=1, keepdims=True) + 1.0
    dinv = lax.rsqrt(deg)
    dr = lax.dot_general(dinv, R, (((0,), (0,)), ((), ())),
                         preferred_element_type=jnp.float32)
    dc = lax.dot_general(dinv, C, (((0,), (0,)), ((), ())),
                         preferred_element_type=jnp.float32)
    Cn = C * (dr * dc)
    A = lax.dot_general(Cn, R, (((1,), (1,)), ((), ())),
                        preferred_element_type=jnp.float32)
    eye = (lax.broadcasted_iota(jnp.int32, (V, V), 0)
           == lax.broadcasted_iota(jnp.int32, (V, V), 1)).astype(jnp.float32)
    a_ref[...] = A + eye * (dinv * dinv)


def _main_kernel(x_ref, w_ref, a_ref, b_ref, o_ref):
    _, O, T, V = o_ref.shape
    C = x_ref.shape[1]
    xm = x_ref[0].reshape(C, T * V)
    y = lax.dot_general(w_ref[...], xm, (((0,), (0,)), ((), ())),
                        preferred_element_type=jnp.float32)
    y3 = y.reshape(O, T, V) + jnp.zeros((1, 1, V), jnp.float32)
    y2 = y3.reshape(O * T, V)
    u = lax.dot_general(y2, a_ref[...], (((1,), (1,)), ((), ())),
                        preferred_element_type=jnp.float32)
    o_ref[0] = u.reshape(O, T, V) + b_ref[...].reshape(O, 1, 1)


def kernel(x, W, b, edge_index):
    n, c, t, v = x.shape
    o = W.shape[1]
    ei = edge_index.astype(jnp.int32)

    A = pl.pallas_call(
        _build_a_kernel,
        out_shape=jax.ShapeDtypeStruct((v, v), jnp.float32),
    )(ei)

    b2 = b.reshape(o, 1)

    out = pl.pallas_call(
        _main_kernel,
        grid=(n,),
        in_specs=[
            pl.BlockSpec((1, c, t, v), lambda i: (i, 0, 0, 0)),
            pl.BlockSpec((c, o), lambda i: (0, 0)),
            pl.BlockSpec((v, v), lambda i: (0, 0)),
            pl.BlockSpec((o, 1), lambda i: (0, 0)),
        ],
        out_specs=pl.BlockSpec((1, o, t, v), lambda i: (i, 0, 0, 0)),
        out_shape=jax.ShapeDtypeStruct((n, o, t, v), jnp.float32),
    )(x, W, A, b2)
    return out


# speedup vs baseline: 77.0781x; 1.4925x over previous
import jax
import jax.numpy as jnp
from jax import lax
from jax.experimental import pallas as pl

_G = 4


def _build_k_kernel(ei_ref, k_ref):
    GV = k_ref.shape[0]
    V = GV // _G
    E = ei_ref.shape[1]
    ei = ei_ref[...]
    row = ei[0:1, :]
    col = ei[1:2, :]
    ids = lax.broadcasted_iota(jnp.int32, (V, E), 0)
    C = (ids == col).astype(jnp.float32)
    R = (ids == row).astype(jnp.float32)
    deg = jnp.sum(C, axis=1, keepdims=True) + 1.0
    dinv = lax.rsqrt(deg)
    dr = lax.dot_general(dinv, R, (((0,), (0,)), ((), ())),
                         preferred_element_type=jnp.float32)
    dc = lax.dot_general(dinv, C, (((0,), (0,)), ((), ())),
                         preferred_element_type=jnp.float32)
    Cn = C * (dr * dc)
    A = lax.dot_general(Cn, R, (((1,), (1,)), ((), ())),
                        preferred_element_type=jnp.float32)
    eye = (lax.broadcasted_iota(jnp.int32, (V, V), 0)
           == lax.broadcasted_iota(jnp.int32, (V, V), 1)).astype(jnp.float32)
    A = A + eye * (dinv * dinv)
    p = lax.broadcasted_iota(jnp.int32, (GV, GV), 0)
    q = lax.broadcasted_iota(jnp.int32, (GV, GV), 1)
    same_block = ((p // V) == (q // V)).astype(jnp.float32)
    Pv = (lax.broadcasted_iota(jnp.int32, (GV, V), 0) % V
          == lax.broadcasted_iota(jnp.int32, (GV, V), 1)).astype(jnp.float32)
    t1 = lax.dot_general(Pv, A, (((1,), (1,)), ((), ())),
                         preferred_element_type=jnp.float32)
    at_big = lax.dot_general(t1, Pv, (((1,), (1,)), ((), ())),
                             preferred_element_type=jnp.float32)
    k_ref[...] = at_big * same_block


def _main_kernel(x_ref, w_ref, k_ref, b_ref, o_ref):
    _, O, TG, GV = o_ref.shape
    y = lax.dot_general(w_ref[...], x_ref[0], (((0,), (0,)), ((), ())),
                        preferred_element_type=jnp.float32)
    y3 = y.reshape(O, TG, GV) + jnp.zeros((1, 1, GV), jnp.float32)
    y2 = y3.reshape(O * TG, GV)
    u = lax.dot_general(y2, k_ref[...], (((1,), (0,)), ((), ())),
                        preferred_element_type=jnp.float32)
    o_ref[0] = u.reshape(O, TG, GV) + b_ref[...].reshape(O, 1, 1)


def kernel(x, W, b, edge_index):
    n, c, t, v = x.shape
    o = W.shape[1]
    ei = edge_index.astype(jnp.int32)
    gv = _G * v
    tg = t // _G

    K = pl.pallas_call(
        _build_k_kernel,
        out_shape=jax.ShapeDtypeStruct((gv, gv), jnp.float32),
    )(ei)

    b2 = b.reshape(o, 1)
    x2 = x.reshape(n, c, t * v)

    out = pl.pallas_call(
        _main_kernel,
        grid=(n,),
        in_specs=[
            pl.BlockSpec((1, c, t * v), lambda i: (i, 0, 0)),
            pl.BlockSpec((c, o), lambda i: (0, 0)),
            pl.BlockSpec((gv, gv), lambda i: (0, 0)),
            pl.BlockSpec((o, 1), lambda i: (0, 0)),
        ],
        out_specs=pl.BlockSpec((1, o, tg, gv), lambda i: (i, 0, 0, 0)),
        out_shape=jax.ShapeDtypeStruct((n, o, tg, gv), jnp.float32),
    )(x2, W, K, b2)
    return out.reshape(n, o, t, v)
